# baseline (speedup 1.0000x reference)
import functools

import jax
import jax.numpy as jnp
from jax import lax
from jax.experimental import pallas as pl
from jax.experimental.pallas import tpu as pltpu
from jax.experimental.pallas import tpu_sc as plsc

NEURONS = 65536
FOCUS = 16
BATCH = 32
FLAT = 262144
OUT_H = 256
OUT_W = 256
LANES = 16
NUM_CORES = 2
NUM_SUBCORES = 16
NW = NUM_CORES * NUM_SUBCORES
NPW = NEURONS // NW
CH = 128
NCHUNK = NPW // CH
CHP = CH + 1

PPW = FLAT // NW
TP = 1024
TCHUNK = PPW // TP
TPP = TP + 1

_PARAMS = pltpu.CompilerParams(use_tc_tiling_on_sc=False,
                               needs_layout_passes=False)


def _make_mesh():
    return plsc.VectorSubcoreMesh(core_axis_name="c", subcore_axis_name="s")


def _make_sc_transpose():
    @functools.partial(
        pl.kernel,
        mesh=_make_mesh(),
        out_type=jax.ShapeDtypeStruct((FLAT, BATCH), jnp.bfloat16),
        scratch_types=[
            pltpu.VMEM((BATCH, TPP), jnp.float32),
            pltpu.VMEM((BATCH, TPP), jnp.float32),
            pltpu.VMEM((TP, BATCH), jnp.bfloat16),
            pltpu.SemaphoreType.DMA,
            pltpu.SemaphoreType.DMA,
        ],
        compiler_params=_PARAMS,
    )
    def sc_transpose(x2, xT, xbuf0, xbuf1, obuf, sem0, sem1):
        wid = lax.axis_index("s") * NUM_CORES + lax.axis_index("c")
        base = wid * PPW
        lane = lax.iota(jnp.int32, LANES)

        def fire(c, xbuf_, sem_):
            p0 = base + c * TP
            pltpu.async_copy(x2.at[:, pl.ds(p0, TP)],
                             xbuf_.at[:, pl.ds(0, TP)], sem_)

        def run(c, xbuf_, sem_):
            pltpu.make_async_copy(x2.at[:, pl.ds(0, TP)],
                                  xbuf_.at[:, pl.ds(0, TP)], sem_).wait()

            @plsc.parallel_loop(0, TP // LANES)
            def group_body(g):
                i0 = g * LANES
                for u in range(LANES):
                    iv = jnp.full((LANES,), i0 + u, jnp.int32)
                    v0 = plsc.load_gather(xbuf_, [lane, iv])
                    v1 = plsc.load_gather(xbuf_, [lane + LANES, iv])
                    packed = plsc.pack(v0, v1,
                                       format=plsc.PackFormat.INTERLEAVED)
                    obuf[i0 + u, :] = packed
            p0 = base + c * TP
            pltpu.sync_copy(obuf, xT.at[pl.ds(p0, TP)])

        fire(0, xbuf0, sem0)

        def pair_body(c2, _):
            c = 2 * c2
            fire(c + 1, xbuf1, sem1)
            run(c, xbuf0, sem0)

            @pl.when(c2 + 1 < TCHUNK // 2)
            def _():
                fire(c + 2, xbuf0, sem0)

            run(c + 1, xbuf1, sem1)
            return 0

        lax.fori_loop(0, TCHUNK // 2, pair_body, 0)

    return sc_transpose


def _make_sc_gather():
    @functools.partial(
        pl.kernel,
        mesh=_make_mesh(),
        out_type=jax.ShapeDtypeStruct((BATCH, NEURONS), jnp.float32),
        scratch_types=[
            pltpu.VMEM((CH * FOCUS,), jnp.int32),
            pltpu.VMEM((CH * FOCUS,), jnp.int32),
            pltpu.VMEM((CH * FOCUS, BATCH), jnp.bfloat16),
            pltpu.VMEM((CH * FOCUS, BATCH), jnp.bfloat16),
            pltpu.VMEM((CH * FOCUS,), jnp.float32),
            pltpu.VMEM((CH * FOCUS,), jnp.float32),
            pltpu.VMEM((CH,), jnp.float32),
            pltpu.VMEM((CH,), jnp.float32),
            pltpu.VMEM((BATCH, CHP), jnp.float32),
            pltpu.VMEM((BATCH, CHP), jnp.float32),
            pltpu.SemaphoreType.DMA,
            pltpu.SemaphoreType.DMA,
            pltpu.SemaphoreType.DMA,
            pltpu.SemaphoreType.DMA,
        ],
        compiler_params=_PARAMS,
    )
    def sc_gather(xT, idx, w, b, out, idxv0, idxv1, gv0, gv1, wv0, wv1,
                  bv0, bv1, ov0, ov1, sem0, sem1, osem0, osem1):
        wid = lax.axis_index("s") * NUM_CORES + lax.axis_index("c")
        base = wid * NPW
        lane = lax.iota(jnp.int32, LANES)

        def fire(c, idxv_, gv_, wv_, bv_, sem_):
            n0 = base + c * CH
            pltpu.sync_copy(idx.at[pl.ds(n0 * FOCUS, CH * FOCUS)], idxv_)
            pltpu.sync_copy(w.at[pl.ds(n0 * FOCUS, CH * FOCUS)], wv_)
            pltpu.sync_copy(b.at[pl.ds(n0, CH)], bv_)
            pltpu.async_copy(xT.at[idxv_], gv_, sem_)

        def run(c, first, idxv_, gv_, wv_, bv_, ov_, sem_, osem_):
            pltpu.make_async_copy(xT.at[idxv_], gv_, sem_).wait()

            @pl.when(jnp.logical_not(first))
            def _():
                pltpu.make_async_copy(ov_.at[:, pl.ds(0, CH)],
                                      out.at[:, pl.ds(0, CH)], osem_).wait()

            @plsc.parallel_loop(0, CH // LANES)
            def group_body(g):
                g0 = g * LANES
                brow = bv_[pl.ds(g0, LANES)]
                for k in range(LANES):
                    j = g0 + k
                    wrow = wv_[pl.ds(j * FOCUS, FOCUS)]
                    acc0 = jnp.full((LANES,), brow[k], jnp.float32)
                    acc1 = acc0
                    r = j * FOCUS
                    for f in range(FOCUS):
                        wf = jnp.full((LANES,), wrow[f], jnp.float32)
                        a0, a1 = plsc.unpack(
                            gv_[r + f, :], format=plsc.PackFormat.INTERLEAVED)
                        acc0 = acc0 + wf * a0
                        acc1 = acc1 + wf * a1
                    col = jnp.full((LANES,), j, jnp.int32)
                    plsc.store_scatter(ov_, [lane, col], acc0)
                    plsc.store_scatter(ov_, [lane + LANES, col], acc1)
            n0 = base + c * CH
            pltpu.async_copy(ov_.at[:, pl.ds(0, CH)],
                             out.at[:, pl.ds(n0, CH)], osem_)

        fire(0, idxv0, gv0, wv0, bv0, sem0)

        def pair_body(c2, _):
            c = 2 * c2
            fire(c + 1, idxv1, gv1, wv1, bv1, sem1)
            run(c, c2 == 0, idxv0, gv0, wv0, bv0, ov0, sem0, osem0)

            @pl.when(c2 + 1 < NCHUNK // 2)
            def _():
                fire(c + 2, idxv0, gv0, wv0, bv0, sem0)

            run(c + 1, c2 == 0, idxv1, gv1, wv1, bv1, ov1, sem1, osem1)
            return 0

        lax.fori_loop(0, NCHUNK // 2, pair_body, 0)
        pltpu.make_async_copy(ov0.at[:, pl.ds(0, CH)],
                              out.at[:, pl.ds(0, CH)], osem0).wait()
        pltpu.make_async_copy(ov1.at[:, pl.ds(0, CH)],
                              out.at[:, pl.ds(0, CH)], osem1).wait()

    return sc_gather


_SC_TRANSPOSE = _make_sc_transpose()
_SC_GATHER = _make_sc_gather()


def kernel(x, weights, bias, connections_index):
    batch = x.shape[0]
    x2 = x.reshape(batch, FLAT)
    xT = _SC_TRANSPOSE(x2)
    idx1 = connections_index.astype(jnp.int32).reshape(-1)
    out = _SC_GATHER(xT, idx1, weights.astype(jnp.float32).reshape(-1),
                     bias.astype(jnp.float32))
    return out.reshape(batch, OUT_H, OUT_W)


# speedup vs baseline: 1.0136x; 1.0136x over previous
import functools

import jax
import jax.numpy as jnp
from jax import lax
from jax.experimental import pallas as pl
from jax.experimental.pallas import tpu as pltpu
from jax.experimental.pallas import tpu_sc as plsc

NEURONS = 65536
FOCUS = 16
BATCH = 32
FLAT = 262144
OUT_H = 256
OUT_W = 256
LANES = 16
NUM_CORES = 2
NUM_SUBCORES = 16
NW = NUM_CORES * NUM_SUBCORES
NPW = NEURONS // NW
CH = 128
NCHUNK = NPW // CH
CHP = CH + 1

PPW = FLAT // NW
TP = 1024
TCHUNK = PPW // TP
TPP = TP + 1

_PARAMS = pltpu.CompilerParams(use_tc_tiling_on_sc=False,
                               needs_layout_passes=False)


def _make_mesh():
    return plsc.VectorSubcoreMesh(core_axis_name="c", subcore_axis_name="s")


def _make_sc_transpose():
    @functools.partial(
        pl.kernel,
        mesh=_make_mesh(),
        out_type=jax.ShapeDtypeStruct((FLAT, BATCH), jnp.bfloat16),
        scratch_types=[
            pltpu.VMEM((BATCH, TPP), jnp.float32),
            pltpu.VMEM((BATCH, TPP), jnp.float32),
            pltpu.VMEM((TP, BATCH), jnp.bfloat16),
            pltpu.SemaphoreType.DMA,
            pltpu.SemaphoreType.DMA,
        ],
        compiler_params=_PARAMS,
    )
    def sc_transpose(x2, xT, xbuf0, xbuf1, obuf, sem0, sem1):
        wid = lax.axis_index("s") * NUM_CORES + lax.axis_index("c")
        base = wid * PPW
        lane = lax.iota(jnp.int32, LANES)

        def fire(c, xbuf_, sem_):
            p0 = base + c * TP
            pltpu.async_copy(x2.at[:, pl.ds(p0, TP)],
                             xbuf_.at[:, pl.ds(0, TP)], sem_)

        def run(c, xbuf_, sem_):
            pltpu.make_async_copy(x2.at[:, pl.ds(0, TP)],
                                  xbuf_.at[:, pl.ds(0, TP)], sem_).wait()

            def group_body(g, _):
                i0 = g * LANES
                for u in range(LANES):
                    iv = jnp.full((LANES,), i0 + u, jnp.int32)
                    v0 = plsc.load_gather(xbuf_, [lane, iv])
                    v1 = plsc.load_gather(xbuf_, [lane + LANES, iv])
                    packed = plsc.pack(v0, v1,
                                       format=plsc.PackFormat.INTERLEAVED)
                    obuf[i0 + u, :] = packed
                return 0

            lax.fori_loop(0, TP // LANES, group_body, 0)
            p0 = base + c * TP
            pltpu.sync_copy(obuf, xT.at[pl.ds(p0, TP)])

        fire(0, xbuf0, sem0)

        def pair_body(c2, _):
            c = 2 * c2
            fire(c + 1, xbuf1, sem1)
            run(c, xbuf0, sem0)

            @pl.when(c2 + 1 < TCHUNK // 2)
            def _():
                fire(c + 2, xbuf0, sem0)

            run(c + 1, xbuf1, sem1)
            return 0

        lax.fori_loop(0, TCHUNK // 2, pair_body, 0)

    return sc_transpose


def _make_sc_gather():
    @functools.partial(
        pl.kernel,
        mesh=_make_mesh(),
        out_type=jax.ShapeDtypeStruct((BATCH, NEURONS), jnp.float32),
        scratch_types=[
            pltpu.VMEM((CH * FOCUS,), jnp.int32),
            pltpu.VMEM((CH * FOCUS,), jnp.int32),
            pltpu.VMEM((CH * FOCUS, BATCH), jnp.bfloat16),
            pltpu.VMEM((CH * FOCUS, BATCH), jnp.bfloat16),
            pltpu.VMEM((CH * FOCUS,), jnp.float32),
            pltpu.VMEM((CH * FOCUS,), jnp.float32),
            pltpu.VMEM((CH,), jnp.float32),
            pltpu.VMEM((CH,), jnp.float32),
            pltpu.VMEM((BATCH, CHP), jnp.float32),
            pltpu.VMEM((BATCH, CHP), jnp.float32),
            pltpu.SemaphoreType.DMA,
            pltpu.SemaphoreType.DMA,
            pltpu.SemaphoreType.DMA,
            pltpu.SemaphoreType.DMA,
        ],
        compiler_params=_PARAMS,
    )
    def sc_gather(xT, idx, w, b, out, idxv0, idxv1, gv0, gv1, wv0, wv1,
                  bv0, bv1, ov0, ov1, sem0, sem1, osem0, osem1):
        wid = lax.axis_index("s") * NUM_CORES + lax.axis_index("c")
        base = wid * NPW
        lane = lax.iota(jnp.int32, LANES)

        def fire(c, idxv_, gv_, wv_, bv_, sem_):
            n0 = base + c * CH
            pltpu.sync_copy(idx.at[pl.ds(n0 * FOCUS, CH * FOCUS)], idxv_)
            pltpu.sync_copy(w.at[pl.ds(n0 * FOCUS, CH * FOCUS)], wv_)
            pltpu.sync_copy(b.at[pl.ds(n0, CH)], bv_)
            pltpu.async_copy(xT.at[idxv_], gv_, sem_)

        def run(c, first, idxv_, gv_, wv_, bv_, ov_, sem_, osem_):
            pltpu.make_async_copy(xT.at[idxv_], gv_, sem_).wait()

            @pl.when(jnp.logical_not(first))
            def _():
                pltpu.make_async_copy(ov_.at[:, pl.ds(0, CH)],
                                      out.at[:, pl.ds(0, CH)], osem_).wait()

            def group_body(g, _):
                g0 = g * LANES
                brow = bv_[pl.ds(g0, LANES)]
                for k in range(LANES):
                    j = g0 + k
                    r = j * FOCUS
                    idxr = jnp.full((LANES,), r, jnp.int32)
                    acc0 = jnp.full((LANES,), brow[k], jnp.float32)
                    acc1 = acc0
                    for f in range(FOCUS):
                        wf = plsc.load_gather(wv_, [idxr + f])
                        a0, a1 = plsc.unpack(
                            gv_[r + f, :], format=plsc.PackFormat.INTERLEAVED)
                        acc0 = acc0 + wf * a0
                        acc1 = acc1 + wf * a1
                    col = jnp.full((LANES,), j, jnp.int32)
                    plsc.store_scatter(ov_, [lane, col], acc0)
                    plsc.store_scatter(ov_, [lane + LANES, col], acc1)
                return 0

            lax.fori_loop(0, CH // LANES, group_body, 0)
            n0 = base + c * CH
            pltpu.async_copy(ov_.at[:, pl.ds(0, CH)],
                             out.at[:, pl.ds(n0, CH)], osem_)

        fire(0, idxv0, gv0, wv0, bv0, sem0)

        def pair_body(c2, _):
            c = 2 * c2
            fire(c + 1, idxv1, gv1, wv1, bv1, sem1)
            run(c, c2 == 0, idxv0, gv0, wv0, bv0, ov0, sem0, osem0)

            @pl.when(c2 + 1 < NCHUNK // 2)
            def _():
                fire(c + 2, idxv0, gv0, wv0, bv0, sem0)

            run(c + 1, c2 == 0, idxv1, gv1, wv1, bv1, ov1, sem1, osem1)
            return 0

        lax.fori_loop(0, NCHUNK // 2, pair_body, 0)
        pltpu.make_async_copy(ov0.at[:, pl.ds(0, CH)],
                              out.at[:, pl.ds(0, CH)], osem0).wait()
        pltpu.make_async_copy(ov1.at[:, pl.ds(0, CH)],
                              out.at[:, pl.ds(0, CH)], osem1).wait()

    return sc_gather


_SC_TRANSPOSE = _make_sc_transpose()
_SC_GATHER = _make_sc_gather()


def kernel(x, weights, bias, connections_index):
    batch = x.shape[0]
    x2 = x.reshape(batch, FLAT)
    xT = _SC_TRANSPOSE(x2)
    idx1 = connections_index.astype(jnp.int32).reshape(-1)
    out = _SC_GATHER(xT, idx1, weights.astype(jnp.float32).reshape(-1),
                     bias.astype(jnp.float32))
    return out.reshape(batch, OUT_H, OUT_W)


# speedup vs baseline: 1.0589x; 1.0447x over previous
import functools

import jax
import jax.numpy as jnp
from jax import lax
from jax.experimental import pallas as pl
from jax.experimental.pallas import tpu as pltpu
from jax.experimental.pallas import tpu_sc as plsc

NEURONS = 65536
FOCUS = 16
BATCH = 32
FLAT = 262144
OUT_H = 256
OUT_W = 256
LANES = 16
NUM_CORES = 2
NUM_SUBCORES = 16
NW = NUM_CORES * NUM_SUBCORES
NPW = NEURONS // NW
CH = 128
NCHUNK = NPW // CH
CHP = CH + 1

PPW = FLAT // NW
TP = 1024
TCHUNK = PPW // TP
TPP = TP + 1

_PARAMS = pltpu.CompilerParams(use_tc_tiling_on_sc=False,
                               needs_layout_passes=False)


def _make_mesh():
    return plsc.VectorSubcoreMesh(core_axis_name="c", subcore_axis_name="s")


def _make_sc_transpose():
    @functools.partial(
        pl.kernel,
        mesh=_make_mesh(),
        out_type=jax.ShapeDtypeStruct((FLAT, BATCH), jnp.bfloat16),
        scratch_types=[
            pltpu.VMEM((BATCH, TPP), jnp.float32),
            pltpu.VMEM((BATCH, TPP), jnp.float32),
            pltpu.VMEM((TP, BATCH), jnp.bfloat16),
            pltpu.VMEM((TP, BATCH), jnp.bfloat16),
            pltpu.SemaphoreType.DMA,
            pltpu.SemaphoreType.DMA,
            pltpu.SemaphoreType.DMA,
            pltpu.SemaphoreType.DMA,
        ],
        compiler_params=_PARAMS,
    )
    def sc_transpose(x2, xT, xbuf0, xbuf1, obuf0, obuf1, sem0, sem1,
                     osem0, osem1):
        wid = lax.axis_index("s") * NUM_CORES + lax.axis_index("c")
        base = wid * PPW
        lane = lax.iota(jnp.int32, LANES)

        def fire(c, xbuf_, sem_):
            p0 = base + c * TP
            pltpu.async_copy(x2.at[:, pl.ds(p0, TP)],
                             xbuf_.at[:, pl.ds(0, TP)], sem_)

        def run(c, first, xbuf_, obuf_, sem_, osem_):
            pltpu.make_async_copy(x2.at[:, pl.ds(0, TP)],
                                  xbuf_.at[:, pl.ds(0, TP)], sem_).wait()

            @pl.when(jnp.logical_not(first))
            def _():
                pltpu.make_async_copy(obuf_, xT.at[pl.ds(0, TP)],
                                      osem_).wait()

            def group_body(g, _):
                i0 = g * LANES
                for u in range(LANES):
                    iv = jnp.full((LANES,), i0 + u, jnp.int32)
                    v0 = plsc.load_gather(xbuf_, [lane, iv])
                    v1 = plsc.load_gather(xbuf_, [lane + LANES, iv])
                    packed = plsc.pack(v0, v1,
                                       format=plsc.PackFormat.INTERLEAVED)
                    obuf_[i0 + u, :] = packed
                return 0

            lax.fori_loop(0, TP // LANES, group_body, 0)
            p0 = base + c * TP
            pltpu.async_copy(obuf_, xT.at[pl.ds(p0, TP)], osem_)

        fire(0, xbuf0, sem0)

        def pair_body(c2, _):
            c = 2 * c2
            fire(c + 1, xbuf1, sem1)
            run(c, c2 == 0, xbuf0, obuf0, sem0, osem0)

            @pl.when(c2 + 1 < TCHUNK // 2)
            def _():
                fire(c + 2, xbuf0, sem0)

            run(c + 1, c2 == 0, xbuf1, obuf1, sem1, osem1)
            return 0

        lax.fori_loop(0, TCHUNK // 2, pair_body, 0)
        pltpu.make_async_copy(obuf0, xT.at[pl.ds(0, TP)], osem0).wait()
        pltpu.make_async_copy(obuf1, xT.at[pl.ds(0, TP)], osem1).wait()

    return sc_transpose


def _make_sc_gather():
    @functools.partial(
        pl.kernel,
        mesh=_make_mesh(),
        out_type=jax.ShapeDtypeStruct((BATCH, NEURONS), jnp.float32),
        scratch_types=[
            pltpu.VMEM((CH * FOCUS,), jnp.int32),
            pltpu.VMEM((CH * FOCUS,), jnp.int32),
            pltpu.VMEM((CH * FOCUS, BATCH), jnp.bfloat16),
            pltpu.VMEM((CH * FOCUS, BATCH), jnp.bfloat16),
            pltpu.VMEM((CH * FOCUS,), jnp.float32),
            pltpu.VMEM((CH * FOCUS,), jnp.float32),
            pltpu.VMEM((CH,), jnp.float32),
            pltpu.VMEM((CH,), jnp.float32),
            pltpu.VMEM((BATCH, CHP), jnp.float32),
            pltpu.VMEM((BATCH, CHP), jnp.float32),
            pltpu.SemaphoreType.DMA,
            pltpu.SemaphoreType.DMA,
            pltpu.SemaphoreType.DMA,
            pltpu.SemaphoreType.DMA,
        ],
        compiler_params=_PARAMS,
    )
    def sc_gather(xT, idx, w, b, out, idxv0, idxv1, gv0, gv1, wv0, wv1,
                  bv0, bv1, ov0, ov1, sem0, sem1, osem0, osem1):
        wid = lax.axis_index("s") * NUM_CORES + lax.axis_index("c")
        base = wid * NPW
        lane = lax.iota(jnp.int32, LANES)

        def fire(c, idxv_, gv_, wv_, bv_, sem_):
            n0 = base + c * CH
            h1 = pltpu.async_copy(idx.at[pl.ds(n0 * FOCUS, CH * FOCUS)],
                                  idxv_, sem_)
            h2 = pltpu.async_copy(w.at[pl.ds(n0 * FOCUS, CH * FOCUS)],
                                  wv_, sem_)
            h3 = pltpu.async_copy(b.at[pl.ds(n0, CH)], bv_, sem_)
            h1.wait()
            h2.wait()
            h3.wait()
            pltpu.async_copy(xT.at[idxv_], gv_, sem_)

        def run(c, first, idxv_, gv_, wv_, bv_, ov_, sem_, osem_):
            pltpu.make_async_copy(xT.at[idxv_], gv_, sem_).wait()

            @pl.when(jnp.logical_not(first))
            def _():
                pltpu.make_async_copy(ov_.at[:, pl.ds(0, CH)],
                                      out.at[:, pl.ds(0, CH)], osem_).wait()

            def group_body(g, _):
                g0 = g * LANES
                brow = bv_[pl.ds(g0, LANES)]
                for k in range(LANES):
                    j = g0 + k
                    r = j * FOCUS
                    idxr = jnp.full((LANES,), r, jnp.int32)
                    acc0 = jnp.full((LANES,), brow[k], jnp.float32)
                    acc1 = acc0
                    for f in range(FOCUS):
                        wf = plsc.load_gather(wv_, [idxr + f])
                        a0, a1 = plsc.unpack(
                            gv_[r + f, :], format=plsc.PackFormat.INTERLEAVED)
                        acc0 = acc0 + wf * a0
                        acc1 = acc1 + wf * a1
                    col = jnp.full((LANES,), j, jnp.int32)
                    plsc.store_scatter(ov_, [lane, col], acc0)
                    plsc.store_scatter(ov_, [lane + LANES, col], acc1)
                return 0

            lax.fori_loop(0, CH // LANES, group_body, 0)
            n0 = base + c * CH
            pltpu.async_copy(ov_.at[:, pl.ds(0, CH)],
                             out.at[:, pl.ds(n0, CH)], osem_)

        fire(0, idxv0, gv0, wv0, bv0, sem0)

        def pair_body(c2, _):
            c = 2 * c2
            fire(c + 1, idxv1, gv1, wv1, bv1, sem1)
            run(c, c2 == 0, idxv0, gv0, wv0, bv0, ov0, sem0, osem0)

            @pl.when(c2 + 1 < NCHUNK // 2)
            def _():
                fire(c + 2, idxv0, gv0, wv0, bv0, sem0)

            run(c + 1, c2 == 0, idxv1, gv1, wv1, bv1, ov1, sem1, osem1)
            return 0

        lax.fori_loop(0, NCHUNK // 2, pair_body, 0)
        pltpu.make_async_copy(ov0.at[:, pl.ds(0, CH)],
                              out.at[:, pl.ds(0, CH)], osem0).wait()
        pltpu.make_async_copy(ov1.at[:, pl.ds(0, CH)],
                              out.at[:, pl.ds(0, CH)], osem1).wait()

    return sc_gather


_SC_TRANSPOSE = _make_sc_transpose()
_SC_GATHER = _make_sc_gather()


def kernel(x, weights, bias, connections_index):
    batch = x.shape[0]
    x2 = x.reshape(batch, FLAT)
    xT = _SC_TRANSPOSE(x2)
    idx1 = connections_index.astype(jnp.int32).reshape(-1)
    out = _SC_GATHER(xT, idx1, weights.astype(jnp.float32).reshape(-1),
                     bias.astype(jnp.float32))
    return out.reshape(batch, OUT_H, OUT_W)
